# Initial kernel scaffold; baseline (speedup 1.0000x reference)
import jax
import jax.numpy as jnp
from jax.experimental import pallas as pl


def kernel(codes_0, codes_1, codes_2, embed_table):
    raise NotImplementedError("write your pallas kernel here")


# speedup vs baseline: 3.0880x; 3.0880x over previous
import functools

import jax
import jax.numpy as jnp
from jax import lax
from jax.experimental import pallas as pl
from jax.experimental.pallas import tpu as pltpu
from jax.experimental.pallas import tpu_sc as plsc

CODEBOOK = 2048
HASH = 1000000
D = 64
L = 16

NC = 2
NS = 16
NW = NC * NS

C = 128


def _body(c0_hbm, c1_hbm, c2_hbm, table_hbm, out_hbm,
          c0_v, c1_v, c2_v, i2_v, i3_v, r1_v, r2_v, r3_v, sem):
    n = out_hbm.shape[0]
    per_w = n // NW
    num_chunks = per_w // C

    wid = lax.axis_index("s") * jnp.int32(NC) + lax.axis_index("c")
    base = wid * jnp.int32(per_w)

    def chunk_body(g, carry):
        off = pl.multiple_of(base + g * jnp.int32(C), C)
        sl = pl.ds(off, C)
        pltpu.sync_copy(c0_hbm.at[sl], c0_v)
        pltpu.sync_copy(c1_hbm.at[sl], c1_v)
        pltpu.sync_copy(c2_hbm.at[sl], c2_v)

        def idx_body(i, carry2):
            s = pl.ds(i * jnp.int32(L), L)
            a0 = c0_v[s]
            a1 = c1_v[s]
            a2 = c2_v[s]
            cb = jnp.int32(CODEBOOK)
            hs = jnp.int32(HASH)
            t2 = lax.rem(a0 * cb + a1, hs)
            t3 = lax.rem(t2 * cb + a2, hs)
            i2_v[s] = t2
            i3_v[s] = t3
            return carry2

        lax.fori_loop(jnp.int32(0), jnp.int32(C // L), idx_body, jnp.int32(0))

        cp1 = pltpu.async_copy(table_hbm.at[c0_v], r1_v, sem)
        cp2 = pltpu.async_copy(table_hbm.at[i2_v], r2_v, sem)
        cp3 = pltpu.async_copy(table_hbm.at[i3_v], r3_v, sem)
        cp1.wait()
        cp2.wait()
        cp3.wait()

        def add_body(r, carry2):
            for j in range(D // L):
                s = pl.ds(j * L, L)
                r1_v[r, s] = r1_v[r, s] + r2_v[r, s] + r3_v[r, s]
            return carry2

        lax.fori_loop(jnp.int32(0), jnp.int32(C), add_body, jnp.int32(0))

        pltpu.sync_copy(r1_v, out_hbm.at[sl])
        return carry

    lax.fori_loop(jnp.int32(0), jnp.int32(num_chunks), chunk_body, jnp.int32(0))


def kernel(codes_0, codes_1, codes_2, embed_table):
    b, h = codes_0.shape
    n = b * h
    c0 = codes_0.reshape(n).astype(jnp.int32)
    c1 = codes_1.reshape(n).astype(jnp.int32)
    c2 = codes_2.reshape(n).astype(jnp.int32)

    mesh = plsc.VectorSubcoreMesh(core_axis_name="c", subcore_axis_name="s")
    run = functools.partial(
        pl.kernel,
        out_type=jax.ShapeDtypeStruct((n, D), jnp.float32),
        mesh=mesh,
        compiler_params=pltpu.CompilerParams(use_tc_tiling_on_sc=False),
        scratch_types=[
            pltpu.VMEM((C,), jnp.int32),
            pltpu.VMEM((C,), jnp.int32),
            pltpu.VMEM((C,), jnp.int32),
            pltpu.VMEM((C,), jnp.int32),
            pltpu.VMEM((C,), jnp.int32),
            pltpu.VMEM((C, D), jnp.float32),
            pltpu.VMEM((C, D), jnp.float32),
            pltpu.VMEM((C, D), jnp.float32),
            pltpu.SemaphoreType.DMA,
        ],
    )(_body)
    out = run(c0, c1, c2, embed_table)
    return out.reshape(b, h, D)


# speedup vs baseline: 4.4286x; 1.4341x over previous
import functools

import jax
import jax.numpy as jnp
from jax import lax
from jax.experimental import pallas as pl
from jax.experimental.pallas import tpu as pltpu
from jax.experimental.pallas import tpu_sc as plsc

CODEBOOK = 2048
HASH = 1000000
D = 64
L = 16

NC = 2
NS = 16
NW = NC * NS

C = 128
NBUF = 2


def _body(c0_hbm, c1_hbm, c2_hbm, table_hbm, out_hbm,
          c0_v, c1_v, c2_v, i2_v, i3_v, r1_v, r2_v, r3_v,
          sem_c, sem_g, sem_out):
    n = out_hbm.shape[0]
    per_w = n // NW
    num_chunks = per_w // C
    g_end = jnp.int32(num_chunks)

    wid = lax.axis_index("s") * jnp.int32(NC) + lax.axis_index("c")
    base = wid * jnp.int32(per_w)

    def chunk_slice(cc):
        off = pl.multiple_of(base + cc * jnp.int32(C), C)
        return pl.ds(off, C)

    def issue_codes(cc, b):
        b = jnp.int32(b)
        sl = chunk_slice(cc)
        pltpu.async_copy(c0_hbm.at[sl], c0_v.at[b], sem_c.at[b])
        pltpu.async_copy(c1_hbm.at[sl], c1_v.at[b], sem_c.at[b])
        pltpu.async_copy(c2_hbm.at[sl], c2_v.at[b], sem_c.at[b])

    def wait_codes(b):
        b = jnp.int32(b)
        sl0 = pl.ds(jnp.int32(0), C)
        pltpu.make_async_copy(c0_hbm.at[sl0], c0_v.at[b], sem_c.at[b]).wait()
        pltpu.make_async_copy(c1_hbm.at[sl0], c1_v.at[b], sem_c.at[b]).wait()
        pltpu.make_async_copy(c2_hbm.at[sl0], c2_v.at[b], sem_c.at[b]).wait()

    def compute_idx(b):
        b = jnp.int32(b)
        def idx_body(i, carry):
            s = pl.ds(i * jnp.int32(L), L)
            a0 = c0_v[b, s]
            a1 = c1_v[b, s]
            a2 = c2_v[b, s]
            cb = jnp.int32(CODEBOOK)
            hs = jnp.int32(HASH)
            t2 = lax.rem(a0 * cb + a1, hs)
            t3 = lax.rem(t2 * cb + a2, hs)
            i2_v[b, s] = t2
            i3_v[b, s] = t3
            return carry

        lax.fori_loop(jnp.int32(0), jnp.int32(C // L), idx_body, jnp.int32(0))

    def issue_gathers(b):
        b = jnp.int32(b)
        pltpu.async_copy(table_hbm.at[c0_v.at[b]], r1_v.at[b], sem_g.at[b])
        pltpu.async_copy(table_hbm.at[i2_v.at[b]], r2_v.at[b], sem_g.at[b])
        pltpu.async_copy(table_hbm.at[i3_v.at[b]], r3_v.at[b], sem_g.at[b])

    def wait_gathers(b):
        b = jnp.int32(b)
        pltpu.make_async_copy(table_hbm.at[c0_v.at[b]], r1_v.at[b],
                              sem_g.at[b]).wait()
        pltpu.make_async_copy(table_hbm.at[i2_v.at[b]], r2_v.at[b],
                              sem_g.at[b]).wait()
        pltpu.make_async_copy(table_hbm.at[i3_v.at[b]], r3_v.at[b],
                              sem_g.at[b]).wait()

    def add_rows(b):
        b = jnp.int32(b)
        def add_body(r, carry):
            for j in range(D // L):
                s = pl.ds(j * L, L)
                r1_v[b, r, s] = r1_v[b, r, s] + r2_v[b, r, s] + r3_v[b, r, s]
            return carry

        lax.fori_loop(jnp.int32(0), jnp.int32(C), add_body, jnp.int32(0))

    def issue_out(cc, b):
        b = jnp.int32(b)
        pltpu.async_copy(r1_v.at[b], out_hbm.at[chunk_slice(cc)],
                         sem_out.at[b])

    def wait_out(b):
        b = jnp.int32(b)
        sl0 = pl.ds(jnp.int32(0), C)
        pltpu.make_async_copy(r1_v.at[b], out_hbm.at[sl0], sem_out.at[b]).wait()

    for b in range(NBUF):
        issue_codes(jnp.int32(b), b)
    wait_codes(0)
    compute_idx(0)
    issue_gathers(0)

    num_rounds = (num_chunks + NBUF - 1) // NBUF

    def round_body(r, carry):
        for b in range(NBUF):
            c = r * jnp.int32(NBUF) + jnp.int32(b)
            nb = (b + 1) % NBUF

            @pl.when(c + 1 < g_end)
            def _head():
                wait_codes(nb)
                compute_idx(nb)

                @pl.when(c + 1 >= jnp.int32(NBUF))
                def _wait_prev_out():
                    wait_out(nb)

                issue_gathers(nb)

            @pl.when(c < g_end)
            def _tail():
                wait_gathers(b)

                @pl.when(c + jnp.int32(NBUF) < g_end)
                def _next_codes():
                    issue_codes(c + jnp.int32(NBUF), b)

                add_rows(b)
                issue_out(c, b)

        return carry

    lax.fori_loop(jnp.int32(0), jnp.int32(num_rounds), round_body,
                  jnp.int32(0))

    for b in range(NBUF):
        wait_out(b)


def kernel(codes_0, codes_1, codes_2, embed_table):
    b, h = codes_0.shape
    n = b * h
    c0 = codes_0.reshape(n).astype(jnp.int32)
    c1 = codes_1.reshape(n).astype(jnp.int32)
    c2 = codes_2.reshape(n).astype(jnp.int32)

    mesh = plsc.VectorSubcoreMesh(core_axis_name="c", subcore_axis_name="s")
    run = functools.partial(
        pl.kernel,
        out_type=jax.ShapeDtypeStruct((n, D), jnp.float32),
        mesh=mesh,
        compiler_params=pltpu.CompilerParams(use_tc_tiling_on_sc=False),
        scratch_types=[
            pltpu.VMEM((NBUF, C), jnp.int32),
            pltpu.VMEM((NBUF, C), jnp.int32),
            pltpu.VMEM((NBUF, C), jnp.int32),
            pltpu.VMEM((NBUF, C), jnp.int32),
            pltpu.VMEM((NBUF, C), jnp.int32),
            pltpu.VMEM((NBUF, C, D), jnp.float32),
            pltpu.VMEM((NBUF, C, D), jnp.float32),
            pltpu.VMEM((NBUF, C, D), jnp.float32),
            pltpu.SemaphoreType.DMA((NBUF,)),
            pltpu.SemaphoreType.DMA((NBUF,)),
            pltpu.SemaphoreType.DMA((NBUF,)),
        ],
    )(_body)
    out = run(c0, c1, c2, embed_table)
    return out.reshape(b, h, D)


# speedup vs baseline: 4.4883x; 1.0135x over previous
import functools

import jax
import jax.numpy as jnp
from jax import lax
from jax.experimental import pallas as pl
from jax.experimental.pallas import tpu as pltpu
from jax.experimental.pallas import tpu_sc as plsc

CODEBOOK = 2048
HASH = 1000000
D = 64
L = 16

NC = 2
NS = 16
NW = NC * NS

C = 128
NBUF = 3


def _mod_ladder(t, ms):
    for m in ms:
        mm = jnp.int32(m)
        t = jnp.where(t >= mm, t - mm, t)
    return t


def _body(c0_hbm, c1_hbm, c2_hbm, table_hbm, out_hbm,
          c0_v, c1_v, c2_v, i2_v, i3_v, r_v,
          sem_c, sem_g1, sem_g23, sem_out):
    n = out_hbm.shape[0]
    per_w = n // NW
    num_chunks = per_w // C
    g_end = jnp.int32(num_chunks)

    wid = lax.axis_index("s") * jnp.int32(NC) + lax.axis_index("c")
    base = wid * jnp.int32(per_w)

    def chunk_slice(cc):
        off = pl.multiple_of(base + cc * jnp.int32(C), C)
        return pl.ds(off, C)

    def issue_codes(cc, b):
        b = jnp.int32(b)
        sl = chunk_slice(cc)
        pltpu.async_copy(c0_hbm.at[sl], c0_v.at[b], sem_c.at[b])
        pltpu.async_copy(c1_hbm.at[sl], c1_v.at[b], sem_c.at[b])
        pltpu.async_copy(c2_hbm.at[sl], c2_v.at[b], sem_c.at[b])

    def wait_codes(b):
        b = jnp.int32(b)
        sl0 = pl.ds(jnp.int32(0), C)
        pltpu.make_async_copy(c0_hbm.at[sl0], c0_v.at[b], sem_c.at[b]).wait()
        pltpu.make_async_copy(c1_hbm.at[sl0], c1_v.at[b], sem_c.at[b]).wait()
        pltpu.make_async_copy(c2_hbm.at[sl0], c2_v.at[b], sem_c.at[b]).wait()

    def compute_idx(b):
        b = jnp.int32(b)

        def idx_body(i, carry):
            s = pl.ds(i * jnp.int32(L), L)
            a0 = c0_v[b, s]
            a1 = c1_v[b, s]
            a2 = c2_v[b, s]
            t2 = _mod_ladder(a0 * jnp.int32(CODEBOOK) + a1,
                             (4000000, 2000000, 1000000))
            hi = lax.shift_right_logical(t2, jnp.int32(10))
            lo = lax.bitwise_and(t2, jnp.int32(1023))
            u = (hi * jnp.int32(97152) + lo * jnp.int32(2048) + a2)
            t3 = _mod_ladder(u, (64000000, 32000000, 16000000, 8000000,
                                 4000000, 2000000, 1000000))
            i2_v[b, s] = t2
            i3_v[b, s] = t3
            return carry

        lax.fori_loop(jnp.int32(0), jnp.int32(C // L), idx_body, jnp.int32(0))

    def issue_g1(b):
        b = jnp.int32(b)
        pltpu.async_copy(table_hbm.at[c0_v.at[b]], r_v.at[b], sem_g1.at[b])

    def wait_g1(b):
        b = jnp.int32(b)
        pltpu.make_async_copy(table_hbm.at[c0_v.at[b]], r_v.at[b],
                              sem_g1.at[b]).wait()

    def issue_g23(b):
        b = jnp.int32(b)
        pltpu.async_copy(table_hbm.at[i2_v.at[b]], r_v.at[b], sem_g23.at[b],
                         add=True)
        pltpu.async_copy(table_hbm.at[i3_v.at[b]], r_v.at[b], sem_g23.at[b],
                         add=True)

    def wait_g23(b):
        b = jnp.int32(b)
        pltpu.make_async_copy(table_hbm.at[i2_v.at[b]], r_v.at[b],
                              sem_g23.at[b]).wait()
        pltpu.make_async_copy(table_hbm.at[i3_v.at[b]], r_v.at[b],
                              sem_g23.at[b]).wait()

    def issue_out(cc, b):
        b = jnp.int32(b)
        pltpu.async_copy(r_v.at[b], out_hbm.at[chunk_slice(cc)], sem_out.at[b])

    def wait_out(b):
        b = jnp.int32(b)
        sl0 = pl.ds(jnp.int32(0), C)
        pltpu.make_async_copy(r_v.at[b], out_hbm.at[sl0], sem_out.at[b]).wait()

    for b in range(NBUF):
        issue_codes(jnp.int32(b), b)
    wait_codes(0)
    compute_idx(0)
    issue_g1(0)
    wait_codes(1)
    compute_idx(1)
    issue_g1(1)
    wait_g1(0)
    issue_g23(0)

    def step_body(c, carry):
        b0 = lax.rem(c, jnp.int32(NBUF))

        @pl.when(c + 2 < g_end)
        def _s1():
            for b in range(NBUF):
                @pl.when(b0 == jnp.int32((b + 1) % NBUF))
                def _do():
                    wait_codes(b)
                    compute_idx(b)

                    @pl.when(c >= jnp.int32(1))
                    def _wait_prev_out():
                        wait_out(b)

                    issue_g1(b)

        @pl.when(c + 1 < g_end)
        def _s2():
            for b in range(NBUF):
                @pl.when(b0 == jnp.int32((b + 2) % NBUF))
                def _do():
                    wait_g1(b)
                    issue_g23(b)

        for b in range(NBUF):
            @pl.when(b0 == jnp.int32(b))
            def _do():
                wait_g23(b)

                @pl.when(c + jnp.int32(NBUF) < g_end)
                def _next_codes():
                    issue_codes(c + jnp.int32(NBUF), b)

                issue_out(c, b)

        return carry

    lax.fori_loop(jnp.int32(0), g_end, step_body, jnp.int32(0))

    for b in range(NBUF):
        wait_out(b)


def kernel(codes_0, codes_1, codes_2, embed_table):
    b, h = codes_0.shape
    n = b * h
    c0 = codes_0.reshape(n).astype(jnp.int32)
    c1 = codes_1.reshape(n).astype(jnp.int32)
    c2 = codes_2.reshape(n).astype(jnp.int32)

    mesh = plsc.VectorSubcoreMesh(core_axis_name="c", subcore_axis_name="s")
    run = functools.partial(
        pl.kernel,
        out_type=jax.ShapeDtypeStruct((n, D), jnp.float32),
        mesh=mesh,
        compiler_params=pltpu.CompilerParams(use_tc_tiling_on_sc=False),
        scratch_types=[
            pltpu.VMEM((NBUF, C), jnp.int32),
            pltpu.VMEM((NBUF, C), jnp.int32),
            pltpu.VMEM((NBUF, C), jnp.int32),
            pltpu.VMEM((NBUF, C), jnp.int32),
            pltpu.VMEM((NBUF, C), jnp.int32),
            pltpu.VMEM((NBUF, C, D), jnp.float32),
            pltpu.SemaphoreType.DMA((NBUF,)),
            pltpu.SemaphoreType.DMA((NBUF,)),
            pltpu.SemaphoreType.DMA((NBUF,)),
            pltpu.SemaphoreType.DMA((NBUF,)),
        ],
    )(_body)
    out = run(c0, c1, c2, embed_table)
    return out.reshape(b, h, D)
